# Initial kernel scaffold; baseline (speedup 1.0000x reference)
import jax
import jax.numpy as jnp
from jax.experimental import pallas as pl


def kernel(q, k_buffer, v_buffer, kv_indptr, kv_indices, num_kv_splits, cos_sin_cache, positions, kv_lora_rank, rotary_dim):
    raise NotImplementedError("write your pallas kernel here")


# speedup vs baseline: 3.6630x; 3.6630x over previous
import functools

import jax
import jax.numpy as jnp
from jax import lax
from jax.experimental import pallas as pl
from jax.experimental.pallas import tpu as pltpu
from jax.experimental.pallas import tpu_sc as plsc

BATCH = 32
H = 16
LORA = 512
ROPE = 64
HEAD = LORA + ROPE
KV = 2048
TOT = BATCH * KV
SPLITS = 8
PER = KV // SPLITS

_NC = 2
_NS = 16
_NW = _NC * _NS
_RPW = TOT // _NW
_CH = 64


def _gather_body(k_hbm, v_hbm, idx_hbm, out_k, out_v, idx_v, kbuf, vbuf,
                 sem_k, sem_v):
    wid = lax.axis_index("s") * _NC + lax.axis_index("c")
    base = wid * _RPW
    pltpu.sync_copy(idx_hbm.at[pl.ds(base, _RPW)], idx_v)

    def chunk(c, carry):
        ck = pltpu.async_copy(k_hbm.at[idx_v.at[pl.ds(c * _CH, _CH)]], kbuf,
                              sem_k)
        cv = pltpu.async_copy(v_hbm.at[idx_v.at[pl.ds(c * _CH, _CH)]], vbuf,
                              sem_v)
        ck.wait()
        pltpu.sync_copy(kbuf, out_k.at[pl.ds(base + c * _CH, _CH)])
        cv.wait()
        pltpu.sync_copy(vbuf, out_v.at[pl.ds(base + c * _CH, _CH)])
        return carry

    lax.fori_loop(0, _RPW // _CH, chunk, 0)


@functools.cache
def _sc_gather():
    return functools.partial(
        pl.kernel,
        out_type=(
            jax.ShapeDtypeStruct((TOT, HEAD), jnp.float32),
            jax.ShapeDtypeStruct((TOT, LORA), jnp.float32),
        ),
        mesh=plsc.VectorSubcoreMesh(core_axis_name="c", subcore_axis_name="s"),
        compiler_params=pltpu.CompilerParams(use_tc_tiling_on_sc=False),
        scratch_types=[
            pltpu.VMEM((_RPW,), jnp.int32),
            pltpu.VMEM((_CH, HEAD), jnp.float32),
            pltpu.VMEM((_CH, LORA), jnp.float32),
            pltpu.SemaphoreType.DMA,
            pltpu.SemaphoreType.DMA,
        ],
    )(_gather_body)


def _flash_body(q_ref, k_ref, v_ref, acc_ref, lse_ref):
    sm_scale = 1.0 / (HEAD ** 0.5)
    q = q_ref[0]
    k = k_ref[...]
    v = v_ref[...]
    qk = lax.dot_general(q, k, (((1,), (1,)), ((), ())),
                         preferred_element_type=jnp.float32) * sm_scale
    m = jnp.max(qk, axis=1, keepdims=True)
    p = jnp.exp(qk - m)
    s = jnp.sum(p, axis=1, keepdims=True)
    acc = lax.dot_general(p, v, (((1,), (0,)), ((), ())),
                          preferred_element_type=jnp.float32)
    acc_ref[0, 0] = acc / s
    lse_ref[0, 0] = m + jnp.log(s)


_tc_flash = pl.pallas_call(
    _flash_body,
    grid=(BATCH, SPLITS),
    in_specs=[
        pl.BlockSpec((1, H, HEAD), lambda b, s: (b, 0, 0)),
        pl.BlockSpec((PER, HEAD), lambda b, s: (b * SPLITS + s, 0)),
        pl.BlockSpec((PER, LORA), lambda b, s: (b * SPLITS + s, 0)),
    ],
    out_specs=[
        pl.BlockSpec((1, 1, H, LORA), lambda b, s: (b, s, 0, 0)),
        pl.BlockSpec((1, 1, H, 1), lambda b, s: (b, s, 0, 0)),
    ],
    out_shape=[
        jax.ShapeDtypeStruct((BATCH, SPLITS, H, LORA), jnp.float32),
        jax.ShapeDtypeStruct((BATCH, SPLITS, H, 1), jnp.float32),
    ],
)


def kernel(q, k_buffer, v_buffer, kv_indptr, kv_indices, num_kv_splits,
           cos_sin_cache, positions, kv_lora_rank, rotary_dim):
    k2d = k_buffer.reshape(TOT, HEAD)
    v2d = v_buffer.reshape(TOT, LORA)
    kx, vx = _sc_gather()(k2d, v2d, kv_indices)
    acc, lse = _tc_flash(q, kx, vx)
    att = jnp.concatenate([acc, lse], axis=-1)
    att = att.transpose(0, 2, 1, 3)
    k_pe_tokens_out = jnp.zeros((1,), dtype=q.dtype)
    return (att, k_pe_tokens_out)


# speedup vs baseline: 4.0149x; 1.0961x over previous
import functools

import jax
import jax.numpy as jnp
from jax import lax
from jax.experimental import pallas as pl
from jax.experimental.pallas import tpu as pltpu
from jax.experimental.pallas import tpu_sc as plsc

BATCH = 32
H = 16
LORA = 512
ROPE = 64
HEAD = LORA + ROPE
KV = 2048
TOT = BATCH * KV
SPLITS = 8
PER = KV // SPLITS

_NC = 2
_NS = 16
_NW = _NC * _NS
_RPW = TOT // _NW
_CH = 64

_RBLK = 4096


def _rope_pack_body(k_ref, out_ref):
    x = k_ref[:, 0, :]
    iota = lax.broadcasted_iota(jnp.int32, (_RBLK, 128), 1)
    out_ref[...] = jnp.where(iota < ROPE, x, 0.0)


_tc_rope_pack = pl.pallas_call(
    _rope_pack_body,
    grid=(TOT // _RBLK,),
    in_specs=[pl.BlockSpec((_RBLK, 1, 128), lambda i: (i, 0, LORA // 128))],
    out_specs=pl.BlockSpec((_RBLK, 128), lambda i: (i, 0)),
    out_shape=jax.ShapeDtypeStruct((TOT, 128), jnp.float32),
)


def _gather_body(k2d, v2d, kr2d, idx_hbm, out_kl, out_kr, out_v,
                 idx_v, klb, krb, vb, sem_kl, sem_kr, sem_v):
    wid = lax.axis_index("s") * _NC + lax.axis_index("c")
    base = wid * _RPW
    pltpu.sync_copy(idx_hbm.at[pl.ds(base, _RPW)], idx_v)

    def chunk(c, carry):
        ixs = idx_v.at[pl.ds(c * _CH, _CH)]
        ckl = pltpu.async_copy(k2d.at[ixs, pl.ds(0, LORA)], klb, sem_kl)
        ckr = pltpu.async_copy(kr2d.at[ixs], krb, sem_kr)
        cv = pltpu.async_copy(v2d.at[ixs], vb, sem_v)
        ckl.wait()
        pltpu.sync_copy(klb, out_kl.at[pl.ds(base + c * _CH, _CH)])
        ckr.wait()
        pltpu.sync_copy(krb, out_kr.at[pl.ds(base + c * _CH, _CH)])
        cv.wait()
        pltpu.sync_copy(vb, out_v.at[pl.ds(base + c * _CH, _CH)])
        return carry

    lax.fori_loop(0, _RPW // _CH, chunk, 0)


@functools.cache
def _sc_gather():
    return functools.partial(
        pl.kernel,
        out_type=(
            jax.ShapeDtypeStruct((TOT, LORA), jnp.float32),
            jax.ShapeDtypeStruct((TOT, 128), jnp.float32),
            jax.ShapeDtypeStruct((TOT, LORA), jnp.float32),
        ),
        mesh=plsc.VectorSubcoreMesh(core_axis_name="c", subcore_axis_name="s"),
        scratch_types=[
            pltpu.VMEM((_RPW,), jnp.int32),
            pltpu.VMEM((_CH, LORA), jnp.float32),
            pltpu.VMEM((_CH, 128), jnp.float32),
            pltpu.VMEM((_CH, LORA), jnp.float32),
            pltpu.SemaphoreType.DMA,
            pltpu.SemaphoreType.DMA,
            pltpu.SemaphoreType.DMA,
        ],
    )(_gather_body)


def _flash_body(ql_ref, qr_ref, kl_ref, kr_ref, v_ref, o_ref):
    sm_scale = 1.0 / (HEAD ** 0.5)
    ql = ql_ref[0]
    qr = qr_ref[0]
    kl = kl_ref[...]
    kr = kr_ref[...]
    v = v_ref[...]
    qk = lax.dot_general(ql, kl, (((1,), (1,)), ((), ())),
                         preferred_element_type=jnp.float32)
    qk = qk + lax.dot_general(qr, kr, (((1,), (1,)), ((), ())),
                              preferred_element_type=jnp.float32)
    qk = qk * sm_scale
    m = jnp.max(qk, axis=1, keepdims=True)
    p = jnp.exp(qk - m)
    s = jnp.sum(p, axis=1, keepdims=True)
    acc = lax.dot_general(p, v, (((1,), (0,)), ((), ())),
                          preferred_element_type=jnp.float32)
    lse = jnp.broadcast_to(m + jnp.log(s), (H, 128))
    o_ref[0, 0] = jnp.concatenate([acc / s, lse], axis=1)


_tc_flash = pl.pallas_call(
    _flash_body,
    grid=(BATCH, SPLITS),
    in_specs=[
        pl.BlockSpec((1, H, LORA), lambda b, s: (b, 0, 0)),
        pl.BlockSpec((1, H, 128), lambda b, s: (b, 0, 0)),
        pl.BlockSpec((PER, LORA), lambda b, s: (b * SPLITS + s, 0)),
        pl.BlockSpec((PER, 128), lambda b, s: (b * SPLITS + s, 0)),
        pl.BlockSpec((PER, LORA), lambda b, s: (b * SPLITS + s, 0)),
    ],
    out_specs=pl.BlockSpec((1, 1, H, LORA + 128), lambda b, s: (b, s, 0, 0)),
    out_shape=jax.ShapeDtypeStruct((BATCH, SPLITS, H, LORA + 128),
                                   jnp.float32),
)


def kernel(q, k_buffer, v_buffer, kv_indptr, kv_indices, num_kv_splits,
           cos_sin_cache, positions, kv_lora_rank, rotary_dim):
    k2d = k_buffer.reshape(TOT, HEAD)
    v2d = v_buffer.reshape(TOT, LORA)
    ql = q[:, :, :LORA]
    qr = jnp.pad(q[:, :, LORA:], ((0, 0), (0, 0), (0, 128 - ROPE)))
    krope = _tc_rope_pack(k_buffer)
    kxl, kxr, vx = _sc_gather()(k2d, v2d, krope, kv_indices)
    out = _tc_flash(ql, qr, kxl, kxr, vx)
    att = out[..., :LORA + 1].transpose(0, 2, 1, 3)
    k_pe_tokens_out = jnp.zeros((1,), dtype=q.dtype)
    return (att, k_pe_tokens_out)
